# Initial kernel scaffold; baseline (speedup 1.0000x reference)
import jax
import jax.numpy as jnp
from jax.experimental import pallas as pl


def kernel(x, edge_index, W1, b1, W2, b2, L1W, L1b, L2W, L2b):
    raise NotImplementedError("write your pallas kernel here")


# speedup vs baseline: 7.9212x; 7.9212x over previous
import functools

import jax
import jax.numpy as jnp
from jax import lax
from jax.experimental import pallas as pl
from jax.experimental.pallas import tpu as pltpu
from jax.experimental.pallas import tpu_sc as plsc

N = 10000
F = 128
E = 320000
K = 128
NCH = 80
NTILES = 32
EPAD = 2 * 16 * NCH * K
NPAD = 10240
RPT = NPAD // 16

_mesh = plsc.VectorSubcoreMesh(core_axis_name="c", subcore_axis_name="s")


@functools.partial(
    pl.kernel, mesh=_mesh,
    out_type=jax.ShapeDtypeStruct((2, NPAD, 16), jnp.float32),
    scratch_types=[
        pltpu.VMEM((NCH, K), jnp.int32),
        pltpu.VMEM((K, 16), jnp.float32),
        pltpu.VMEM((K, 16), jnp.float32),
        pltpu.VMEM_SHARED((NPAD, 16), jnp.float32),
    ],
)
def _deg_kernel(dst_hbm, out_hbm, dst_v, ones_v, zeros_v, acc_sh):
    c = lax.axis_index("c")
    s = lax.axis_index("s")

    def fill(i, carry):
        ones_v[i, :] = jnp.ones((16,), jnp.float32)
        zeros_v[i, :] = jnp.zeros((16,), jnp.float32)
        return carry

    lax.fori_loop(0, K, fill, 0)

    def zslice(i, carry):
        pltpu.sync_copy(zeros_v, acc_sh.at[pl.ds(s * RPT + i * K, K)])
        return carry

    lax.fori_loop(0, RPT // K, zslice, 0)
    pltpu.sync_copy(dst_hbm.at[c, s], dst_v)
    plsc.subcore_barrier()

    def chunk(j, carry):
        pltpu.sync_copy(ones_v, acc_sh.at[dst_v.at[j]], add=True)
        return carry

    lax.fori_loop(0, NCH, chunk, 0)
    plsc.subcore_barrier()
    pltpu.sync_copy(acc_sh.at[pl.ds(s * RPT, RPT)],
                    out_hbm.at[c, pl.ds(s * RPT, RPT)])


NQ = 2
QCH = NCH // NQ


@functools.partial(
    pl.kernel, mesh=_mesh,
    out_type=jax.ShapeDtypeStruct((2, NPAD, F), jnp.float32),
    scratch_types=[
        pltpu.VMEM((QCH, K), jnp.int32),
        pltpu.VMEM((QCH, K), jnp.int32),
        pltpu.VMEM((K, F), jnp.float32),
        pltpu.VMEM((K, F), jnp.float32),
        pltpu.VMEM_SHARED((NPAD, F), jnp.float32),
        pltpu.SemaphoreType.DMA,
        pltpu.SemaphoreType.DMA,
    ],
)
def _prop_kernel(src_hbm, dst_hbm, table_hbm, out_hbm,
                 src_v, dst_v, rows_a, rows_b, acc_sh, sem_a, sem_b):
    c = lax.axis_index("c")
    s = lax.axis_index("s")

    def zrow(i, carry):
        for l in range(F // 16):
            rows_a[i, pl.ds(l * 16, 16)] = jnp.zeros((16,), jnp.float32)
        return carry

    lax.fori_loop(0, K, zrow, 0)

    def zslice(i, carry):
        pltpu.sync_copy(rows_a, acc_sh.at[pl.ds(s * RPT + i * K, K)])
        return carry

    lax.fori_loop(0, RPT // K, zslice, 0)
    plsc.subcore_barrier()

    def phase(q, carry):
        pltpu.sync_copy(src_hbm.at[c, s, pl.ds(q * QCH, QCH)], src_v)
        pltpu.sync_copy(dst_hbm.at[c, s, pl.ds(q * QCH, QCH)], dst_v)
        pltpu.async_copy(table_hbm.at[src_v.at[0]], rows_a, sem_a)
        pltpu.async_copy(table_hbm.at[src_v.at[1]], rows_b, sem_b)

        def pair(i, carry2):
            j = 2 * i
            pltpu.make_async_copy(table_hbm.at[src_v.at[j]],
                                  rows_a, sem_a).wait()
            pltpu.sync_copy(rows_a, acc_sh.at[dst_v.at[j]], add=True)
            pltpu.async_copy(table_hbm.at[src_v.at[j + 2]], rows_a, sem_a)
            pltpu.make_async_copy(table_hbm.at[src_v.at[j + 1]],
                                  rows_b, sem_b).wait()
            pltpu.sync_copy(rows_b, acc_sh.at[dst_v.at[j + 1]], add=True)
            pltpu.async_copy(table_hbm.at[src_v.at[j + 3]], rows_b, sem_b)
            return carry2

        lax.fori_loop(0, QCH // 2 - 1, pair, 0)
        pltpu.make_async_copy(table_hbm.at[src_v.at[QCH - 2]],
                              rows_a, sem_a).wait()
        pltpu.sync_copy(rows_a, acc_sh.at[dst_v.at[QCH - 2]], add=True)
        pltpu.make_async_copy(table_hbm.at[src_v.at[QCH - 1]],
                              rows_b, sem_b).wait()
        pltpu.sync_copy(rows_b, acc_sh.at[dst_v.at[QCH - 1]], add=True)
        return carry

    lax.fori_loop(0, NQ, phase, 0)

    plsc.subcore_barrier()
    pltpu.sync_copy(acc_sh.at[pl.ds(s * RPT, RPT)],
                    out_hbm.at[c, pl.ds(s * RPT, RPT)])


RB = 2000


def _prep1_body(x_ref, dA_ref, dB_ref, W_ref, s1_ref, self1_ref, disb_ref):
    dis = lax.rsqrt(1.0 + dA_ref[:, 0:1] + dB_ref[:, 0:1])
    xw = jnp.dot(x_ref[...], W_ref[...], preferred_element_type=jnp.float32)
    s1 = xw * dis
    s1_ref[...] = s1
    self1_ref[...] = s1 * dis
    disb_ref[...] = jnp.broadcast_to(dis, s1.shape)


def _prep1(x, degA, degB, W1):
    return pl.pallas_call(
        _prep1_body,
        grid=(N // RB,),
        in_specs=[
            pl.BlockSpec((RB, F), lambda i: (i, 0)),
            pl.BlockSpec((RB, 16), lambda i: (i, 0)),
            pl.BlockSpec((RB, 16), lambda i: (i, 0)),
            pl.BlockSpec((F, F), lambda i: (0, 0)),
        ],
        out_specs=[pl.BlockSpec((RB, F), lambda i: (i, 0))] * 3,
        out_shape=[jax.ShapeDtypeStruct((N, F), jnp.float32)] * 3,
    )(x, degA, degB, W1)


def _mid_body(aA_ref, aB_ref, self1_ref, disb_ref, b1_ref, W_ref,
              x1_ref, s2_ref, self2_ref):
    dis = disb_ref[...]
    x1 = jnp.maximum(
        dis * (aA_ref[...] + aB_ref[...]) + self1_ref[...] + b1_ref[...], 0.0)
    xw2 = jnp.dot(x1, W_ref[...], preferred_element_type=jnp.float32)
    s2 = xw2 * dis
    x1_ref[...] = x1
    s2_ref[...] = s2
    self2_ref[...] = s2 * dis


def _mid(aggA, aggB, self1, disb, b1, W2):
    return pl.pallas_call(
        _mid_body,
        grid=(N // RB,),
        in_specs=[
            pl.BlockSpec((RB, F), lambda i: (i, 0)),
            pl.BlockSpec((RB, F), lambda i: (i, 0)),
            pl.BlockSpec((RB, F), lambda i: (i, 0)),
            pl.BlockSpec((RB, F), lambda i: (i, 0)),
            pl.BlockSpec((1, F), lambda i: (0, 0)),
            pl.BlockSpec((F, F), lambda i: (0, 0)),
        ],
        out_specs=[pl.BlockSpec((RB, F), lambda i: (i, 0))] * 3,
        out_shape=[jax.ShapeDtypeStruct((N, F), jnp.float32)] * 3,
    )(aggA, aggB, self1, disb, b1, W2)


def _head_body(aA_ref, aB_ref, self2_ref, disb_ref, b2_ref, x_ref, x1_ref,
               L1W_ref, L1b_ref, L2W_ref, L2b_ref, out_ref):
    dis = disb_ref[...]
    x2 = jnp.maximum(
        dis * (aA_ref[...] + aB_ref[...]) + self2_ref[...] + b2_ref[...], 0.0)
    t = (jnp.dot(x_ref[...], L1W_ref[0:F, :], preferred_element_type=jnp.float32)
         + jnp.dot(x1_ref[...], L1W_ref[F:2 * F, :],
                   preferred_element_type=jnp.float32)
         + jnp.dot(x2, L1W_ref[2 * F:3 * F, :],
                   preferred_element_type=jnp.float32)
         + L1b_ref[...])
    t = jnp.maximum(t, 0.0)
    z = jnp.dot(t, L2W_ref[...], preferred_element_type=jnp.float32) + L2b_ref[...]
    out_ref[...] = jax.nn.sigmoid(z)


def _head(aggA, aggB, self2, disb, b2, x, x1, L1W, L1b, L2W, L2b):
    return pl.pallas_call(
        _head_body,
        grid=(N // RB,),
        in_specs=[
            pl.BlockSpec((RB, F), lambda i: (i, 0)),
            pl.BlockSpec((RB, F), lambda i: (i, 0)),
            pl.BlockSpec((RB, F), lambda i: (i, 0)),
            pl.BlockSpec((RB, F), lambda i: (i, 0)),
            pl.BlockSpec((1, F), lambda i: (0, 0)),
            pl.BlockSpec((RB, F), lambda i: (i, 0)),
            pl.BlockSpec((RB, F), lambda i: (i, 0)),
            pl.BlockSpec((3 * F, F), lambda i: (0, 0)),
            pl.BlockSpec((1, F), lambda i: (0, 0)),
            pl.BlockSpec((F, 64), lambda i: (0, 0)),
            pl.BlockSpec((1, 64), lambda i: (0, 0)),
        ],
        out_specs=pl.BlockSpec((RB, 64), lambda i: (i, 0)),
        out_shape=jax.ShapeDtypeStruct((N, 64), jnp.float32),
    )(aggA, aggB, self2, disb, b2, x, x1, L1W, L1b, L2W, L2b)


def kernel(x, edge_index, W1, b1, W2, b2, L1W, L1b, L2W, L2b):
    src = edge_index[0].astype(jnp.int32)
    dst = edge_index[1].astype(jnp.int32)
    pad = EPAD - E
    src_p = jnp.concatenate([src, jnp.zeros((pad,), jnp.int32)])
    dst_p = jnp.concatenate([dst, jnp.full((pad,), N, jnp.int32)])
    src_p = src_p.reshape(2, 16, NCH, K)
    dst_p = dst_p.reshape(2, 16, NCH, K)

    degs = _deg_kernel(dst_p)
    degA = degs[0, :N, :]
    degB = degs[1, :N, :]

    s1, self1, disb = _prep1(x, degA, degB, W1)
    agg1 = _prop_kernel(src_p, dst_p, s1)
    x1, s2, self2 = _mid(agg1[0, :N], agg1[1, :N], self1, disb,
                         b1.reshape(1, F), W2)
    agg2 = _prop_kernel(src_p, dst_p, s2)
    out = _head(agg2[0, :N], agg2[1, :N], self2, disb, b2.reshape(1, F),
                x, x1, L1W, L1b.reshape(1, F), L2W, L2b.reshape(1, 64))
    return out


# speedup vs baseline: 10.4888x; 1.3241x over previous
import functools

import jax
import jax.numpy as jnp
from jax import lax
from jax.experimental import pallas as pl
from jax.experimental.pallas import tpu as pltpu
from jax.experimental.pallas import tpu_sc as plsc

N = 10000
F = 128
E = 320000
K = 128
NCH = 80
NTILES = 32
EPAD = 2 * 16 * NCH * K
NPAD = 10240
RPT = NPAD // 16

_mesh = plsc.VectorSubcoreMesh(core_axis_name="c", subcore_axis_name="s")


@functools.partial(
    pl.kernel, mesh=_mesh,
    out_type=jax.ShapeDtypeStruct((2, NPAD, 16), jnp.float32),
    scratch_types=[
        pltpu.VMEM((NCH, K), jnp.int32),
        pltpu.VMEM((K, 16), jnp.float32),
        pltpu.VMEM((K, 16), jnp.float32),
        pltpu.VMEM_SHARED((NPAD, 16), jnp.float32),
    ],
)
def _deg_kernel(dst_hbm, out_hbm, dst_v, ones_v, zeros_v, acc_sh):
    c = lax.axis_index("c")
    s = lax.axis_index("s")

    def fill(i, carry):
        ones_v[i, :] = jnp.ones((16,), jnp.float32)
        zeros_v[i, :] = jnp.zeros((16,), jnp.float32)
        return carry

    lax.fori_loop(0, K, fill, 0)

    def zslice(i, carry):
        pltpu.sync_copy(zeros_v, acc_sh.at[pl.ds(s * RPT + i * K, K)])
        return carry

    lax.fori_loop(0, RPT // K, zslice, 0)
    w = pl.multiple_of((c * 16 + s) * NCH, 16)
    pltpu.sync_copy(dst_hbm.at[pl.ds(w, NCH)], dst_v)
    plsc.subcore_barrier()

    def chunk(j, carry):
        pltpu.sync_copy(ones_v, acc_sh.at[dst_v.at[j]], add=True)
        return carry

    lax.fori_loop(0, NCH, chunk, 0)
    plsc.subcore_barrier()
    pltpu.sync_copy(acc_sh.at[pl.ds(s * RPT, RPT)],
                    out_hbm.at[c, pl.ds(s * RPT, RPT)])


NCH0 = 128
NCH1 = 32
NCHT = 16 * (NCH0 + NCH1)
QCH = 32


@functools.partial(
    pl.kernel, mesh=_mesh,
    out_type=jax.ShapeDtypeStruct((2, NPAD, F), jnp.float32),
    scratch_types=[
        pltpu.VMEM((QCH, K), jnp.int32),
        pltpu.VMEM((QCH, K), jnp.int32),
        pltpu.VMEM((K, F), jnp.float32),
        pltpu.VMEM((K, F), jnp.float32),
        pltpu.VMEM_SHARED((NPAD, F), jnp.float32),
        pltpu.SemaphoreType.DMA,
        pltpu.SemaphoreType.DMA,
    ],
)
def _prop_kernel(src_hbm, dst_hbm, table_hbm, out_hbm,
                 src_v, dst_v, rows_a, rows_b, acc_sh, sem_a, sem_b):
    c = lax.axis_index("c")
    s = lax.axis_index("s")
    nph = jnp.where(c == 0, NCH0 // QCH, NCH1 // QCH)
    base = jnp.where(c == 0, s * NCH0, 16 * NCH0 + s * NCH1)

    def zrow(i, carry):
        for l in range(F // 16):
            rows_a[i, pl.ds(l * 16, 16)] = jnp.zeros((16,), jnp.float32)
        return carry

    lax.fori_loop(0, K, zrow, 0)

    def zslice(i, carry):
        pltpu.sync_copy(rows_a, acc_sh.at[pl.ds(s * RPT + i * K, K)])
        return carry

    lax.fori_loop(0, RPT // K, zslice, 0)
    plsc.subcore_barrier()

    def phase(q, carry):
        off = pl.multiple_of(base + q * QCH, QCH)
        pltpu.sync_copy(src_hbm.at[pl.ds(off, QCH)], src_v)
        pltpu.sync_copy(dst_hbm.at[pl.ds(off, QCH)], dst_v)
        pltpu.async_copy(table_hbm.at[src_v.at[0]], rows_a, sem_a)
        pltpu.async_copy(table_hbm.at[src_v.at[1]], rows_b, sem_b)

        def pair(i, carry2):
            j = 2 * i
            pltpu.make_async_copy(table_hbm.at[src_v.at[j]],
                                  rows_a, sem_a).wait()
            pltpu.sync_copy(rows_a, acc_sh.at[dst_v.at[j]], add=True)
            pltpu.async_copy(table_hbm.at[src_v.at[j + 2]], rows_a, sem_a)
            pltpu.make_async_copy(table_hbm.at[src_v.at[j + 1]],
                                  rows_b, sem_b).wait()
            pltpu.sync_copy(rows_b, acc_sh.at[dst_v.at[j + 1]], add=True)
            pltpu.async_copy(table_hbm.at[src_v.at[j + 3]], rows_b, sem_b)
            return carry2

        lax.fori_loop(0, QCH // 2 - 1, pair, 0)
        pltpu.make_async_copy(table_hbm.at[src_v.at[QCH - 2]],
                              rows_a, sem_a).wait()
        pltpu.sync_copy(rows_a, acc_sh.at[dst_v.at[QCH - 2]], add=True)
        pltpu.make_async_copy(table_hbm.at[src_v.at[QCH - 1]],
                              rows_b, sem_b).wait()
        pltpu.sync_copy(rows_b, acc_sh.at[dst_v.at[QCH - 1]], add=True)
        return carry

    lax.fori_loop(0, nph, phase, 0)

    plsc.subcore_barrier()
    pltpu.sync_copy(acc_sh.at[pl.ds(s * RPT, RPT)],
                    out_hbm.at[c, pl.ds(s * RPT, RPT)])


RB = 2000


def _prep1_body(x_ref, dA_ref, dB_ref, W_ref, s1_ref, self1_ref, disb_ref):
    dis = lax.rsqrt(1.0 + dA_ref[:, 0:1] + dB_ref[:, 0:1])
    xw = jnp.dot(x_ref[...], W_ref[...], preferred_element_type=jnp.float32)
    s1 = xw * dis
    s1_ref[...] = s1
    self1_ref[...] = s1 * dis
    disb_ref[...] = jnp.broadcast_to(dis, s1.shape)


def _prep1(x, degA, degB, W1):
    return pl.pallas_call(
        _prep1_body,
        grid=(N // RB,),
        in_specs=[
            pl.BlockSpec((RB, F), lambda i: (i, 0)),
            pl.BlockSpec((RB, 16), lambda i: (i, 0)),
            pl.BlockSpec((RB, 16), lambda i: (i, 0)),
            pl.BlockSpec((F, F), lambda i: (0, 0)),
        ],
        out_specs=[pl.BlockSpec((RB, F), lambda i: (i, 0))] * 3,
        out_shape=[jax.ShapeDtypeStruct((N, F), jnp.float32)] * 3,
    )(x, degA, degB, W1)


def _mid_body(aA_ref, aB_ref, self1_ref, disb_ref, b1_ref, W_ref,
              x1_ref, s2_ref, self2_ref):
    dis = disb_ref[...]
    x1 = jnp.maximum(
        dis * (aA_ref[...] + aB_ref[...]) + self1_ref[...] + b1_ref[...], 0.0)
    xw2 = jnp.dot(x1, W_ref[...], preferred_element_type=jnp.float32)
    s2 = xw2 * dis
    x1_ref[...] = x1
    s2_ref[...] = s2
    self2_ref[...] = s2 * dis


def _mid(aggA, aggB, self1, disb, b1, W2):
    return pl.pallas_call(
        _mid_body,
        grid=(N // RB,),
        in_specs=[
            pl.BlockSpec((RB, F), lambda i: (i, 0)),
            pl.BlockSpec((RB, F), lambda i: (i, 0)),
            pl.BlockSpec((RB, F), lambda i: (i, 0)),
            pl.BlockSpec((RB, F), lambda i: (i, 0)),
            pl.BlockSpec((1, F), lambda i: (0, 0)),
            pl.BlockSpec((F, F), lambda i: (0, 0)),
        ],
        out_specs=[pl.BlockSpec((RB, F), lambda i: (i, 0))] * 3,
        out_shape=[jax.ShapeDtypeStruct((N, F), jnp.float32)] * 3,
    )(aggA, aggB, self1, disb, b1, W2)


def _head_body(aA_ref, aB_ref, self2_ref, disb_ref, b2_ref, x_ref, x1_ref,
               L1W_ref, L1b_ref, L2W_ref, L2b_ref, out_ref):
    dis = disb_ref[...]
    x2 = jnp.maximum(
        dis * (aA_ref[...] + aB_ref[...]) + self2_ref[...] + b2_ref[...], 0.0)
    t = (jnp.dot(x_ref[...], L1W_ref[0:F, :], preferred_element_type=jnp.float32)
         + jnp.dot(x1_ref[...], L1W_ref[F:2 * F, :],
                   preferred_element_type=jnp.float32)
         + jnp.dot(x2, L1W_ref[2 * F:3 * F, :],
                   preferred_element_type=jnp.float32)
         + L1b_ref[...])
    t = jnp.maximum(t, 0.0)
    z = jnp.dot(t, L2W_ref[...], preferred_element_type=jnp.float32) + L2b_ref[...]
    out_ref[...] = jax.nn.sigmoid(z)


def _head(aggA, aggB, self2, disb, b2, x, x1, L1W, L1b, L2W, L2b):
    return pl.pallas_call(
        _head_body,
        grid=(N // RB,),
        in_specs=[
            pl.BlockSpec((RB, F), lambda i: (i, 0)),
            pl.BlockSpec((RB, F), lambda i: (i, 0)),
            pl.BlockSpec((RB, F), lambda i: (i, 0)),
            pl.BlockSpec((RB, F), lambda i: (i, 0)),
            pl.BlockSpec((1, F), lambda i: (0, 0)),
            pl.BlockSpec((RB, F), lambda i: (i, 0)),
            pl.BlockSpec((RB, F), lambda i: (i, 0)),
            pl.BlockSpec((3 * F, F), lambda i: (0, 0)),
            pl.BlockSpec((1, F), lambda i: (0, 0)),
            pl.BlockSpec((F, 64), lambda i: (0, 0)),
            pl.BlockSpec((1, 64), lambda i: (0, 0)),
        ],
        out_specs=pl.BlockSpec((RB, 64), lambda i: (i, 0)),
        out_shape=jax.ShapeDtypeStruct((N, 64), jnp.float32),
    )(aggA, aggB, self2, disb, b2, x, x1, L1W, L1b, L2W, L2b)


def kernel(x, edge_index, W1, b1, W2, b2, L1W, L1b, L2W, L2b):
    src = edge_index[0].astype(jnp.int32)
    dst = edge_index[1].astype(jnp.int32)
    pad = EPAD - E
    src_p = jnp.concatenate([src, jnp.zeros((pad,), jnp.int32)])
    dst_p = jnp.concatenate(
        [dst, N + jnp.arange(pad, dtype=jnp.int32) % (NPAD - N)])
    src_p = src_p.reshape(NCHT, K)
    dst_p = dst_p.reshape(NCHT, K)

    degs = _deg_kernel(dst_p)
    degA = degs[0, :N, :]
    degB = degs[1, :N, :]

    s1, self1, disb = _prep1(x, degA, degB, W1)
    agg1 = _prop_kernel(src_p, dst_p, s1)
    x1, s2, self2 = _mid(agg1[0, :N], agg1[1, :N], self1, disb,
                         b1.reshape(1, F), W2)
    agg2 = _prop_kernel(src_p, dst_p, s2)
    out = _head(agg2[0, :N], agg2[1, :N], self2, disb, b2.reshape(1, F),
                x, x1, L1W, L1b.reshape(1, F), L2W, L2b.reshape(1, 64))
    return out
